# baseline (speedup 1.0000x reference)
import functools

import jax
import jax.numpy as jnp
from jax import lax
from jax.experimental import pallas as pl
from jax.experimental.pallas import tpu as pltpu
from jax.experimental.pallas import tpu_sc as plsc

B = 16384
D = 64

_info = plsc.get_sparse_core_info()
_NC, _NS, _L = _info.num_cores, _info.num_subcores, _info.num_lanes
_NW = _NC * _NS
_BPW = B // _NW
_CH = 128
_NCH = _BPW // _CH


def _lane_take(x, idx):
    dnums = lax.GatherDimensionNumbers(
        offset_dims=(), collapsed_slice_dims=(0,), start_index_map=(0,))
    return lax.gather(x, idx[:, None], dnums, (1,),
                      mode=lax.GatherScatterMode.PROMISE_IN_BOUNDS)


def _normalize_rows(rows_ref, n_rows):
    lanes = lax.iota(jnp.int32, _L)
    perms = [lanes ^ sh for sh in (8, 4, 2, 1)]

    def body(r, carry):
        v0 = rows_ref[r, pl.ds(0, _L)]
        v1 = rows_ref[r, pl.ds(_L, _L)]
        v2 = rows_ref[r, pl.ds(2 * _L, _L)]
        v3 = rows_ref[r, pl.ds(3 * _L, _L)]
        ss = v0 * v0 + v1 * v1 + v2 * v2 + v3 * v3
        for p in perms:
            ss = ss + _lane_take(ss, p)
        s = ss
        i = lax.bitcast_convert_type(s, jnp.int32)
        y = lax.bitcast_convert_type(0x5F3759DF - (i >> 1), jnp.float32)
        nhalf = s * (-0.5)
        for _ in range(3):
            y = y * (1.5 + nhalf * y * y)
        rows_ref[r, pl.ds(0, _L)] = v0 * y
        rows_ref[r, pl.ds(_L, _L)] = v1 * y
        rows_ref[r, pl.ds(2 * _L, _L)] = v2 * y
        rows_ref[r, pl.ds(3 * _L, _L)] = v3 * y
        return carry

    lax.fori_loop(0, n_rows, body, 0)


def _sc_body(
    u_idx_hbm, p_idx_hbm, n_idx_hbm,
    user_table, item_table, user_bias, item_bias,
    out_ue, out_pe, out_ne, out_ub, out_pb, out_nb,
    rows_u, rows_p, rows_n,
    vidx_u, vidx_p, vidx_n,
    sem_u, sem_p, sem_n, sem_b,
):
    wid = lax.axis_index("s") * _NC + lax.axis_index("c")
    base = wid * _BPW

    pltpu.sync_copy(u_idx_hbm.at[pl.ds(base, _BPW)], vidx_u)
    pltpu.sync_copy(p_idx_hbm.at[pl.ds(base, _BPW)], vidx_p)
    pltpu.sync_copy(n_idx_hbm.at[pl.ds(base, _BPW)], vidx_n)

    def issue_bias(g, carry):
        gb = g * _L
        vu = vidx_u[pl.ds(gb, _L)]
        vp = vidx_p[pl.ds(gb, _L)]
        vn = vidx_n[pl.ds(gb, _L)]
        for l in range(_L):
            dst = pl.ds(base + gb + l, 1)
            pltpu.async_copy(user_bias.at[pl.ds(vu[l], 1)],
                             out_ub.at[dst], sem_b)
            pltpu.async_copy(item_bias.at[pl.ds(vp[l], 1)],
                             out_pb.at[dst], sem_b)
            pltpu.async_copy(item_bias.at[pl.ds(vn[l], 1)],
                             out_nb.at[dst], sem_b)
        return carry

    lax.fori_loop(0, _BPW // _L, issue_bias, 0)

    def issue_chunk(c):
        def issue(g, carry):
            gb = g * _L
            vu = vidx_u[pl.ds(c * _CH + gb, _L)]
            vp = vidx_p[pl.ds(c * _CH + gb, _L)]
            vn = vidx_n[pl.ds(c * _CH + gb, _L)]
            for l in range(_L):
                dst = pl.ds(gb + l, 1)
                pltpu.async_copy(user_table.at[pl.ds(vu[l], 1)],
                                 rows_u.at[dst], sem_u)
                pltpu.async_copy(item_table.at[pl.ds(vp[l], 1)],
                                 rows_p.at[dst], sem_p)
                pltpu.async_copy(item_table.at[pl.ds(vn[l], 1)],
                                 rows_n.at[dst], sem_n)
            return carry
        lax.fori_loop(0, _CH // _L, issue, 0)

    def drain_chunk():
        pltpu.make_async_copy(
            user_table.at[pl.ds(0, _CH)], rows_u, sem_u).wait()
        pltpu.make_async_copy(
            item_table.at[pl.ds(0, _CH)], rows_p, sem_p).wait()
        pltpu.make_async_copy(
            item_table.at[pl.ds(0, _CH)], rows_n, sem_n).wait()

    for c in range(_NCH):
        issue_chunk(c)
        drain_chunk()
        _normalize_rows(rows_u, _CH)
        _normalize_rows(rows_p, _CH)
        _normalize_rows(rows_n, _CH)
        odst = pl.ds(base + c * _CH, _CH)
        pltpu.sync_copy(rows_u, out_ue.at[odst])
        pltpu.sync_copy(rows_p, out_pe.at[odst])
        pltpu.sync_copy(rows_n, out_ne.at[odst])

    pltpu.make_async_copy(
        user_bias.at[pl.ds(0, _BPW)], out_ub.at[pl.ds(base, _BPW)],
        sem_b).wait()
    pltpu.make_async_copy(
        item_bias.at[pl.ds(0, _BPW)], out_pb.at[pl.ds(base, _BPW)],
        sem_b).wait()
    pltpu.make_async_copy(
        item_bias.at[pl.ds(0, _BPW)], out_nb.at[pl.ds(base, _BPW)],
        sem_b).wait()


@jax.jit
def _bpr_lookup(user, pos, neg, user_table, item_table,
                user_bias_table, item_bias_table):
    mesh = plsc.VectorSubcoreMesh(core_axis_name="c", subcore_axis_name="s")
    f32 = jnp.float32
    call = functools.partial(
        pl.kernel,
        mesh=mesh,
        compiler_params=pltpu.CompilerParams(use_tc_tiling_on_sc=True),
        out_type=[
            jax.ShapeDtypeStruct((B, D), f32),
            jax.ShapeDtypeStruct((B, D), f32),
            jax.ShapeDtypeStruct((B, D), f32),
            jax.ShapeDtypeStruct((B, 1), f32),
            jax.ShapeDtypeStruct((B, 1), f32),
            jax.ShapeDtypeStruct((B, 1), f32),
        ],
        scratch_types=[
            pltpu.VMEM((_CH, D), f32),
            pltpu.VMEM((_CH, D), f32),
            pltpu.VMEM((_CH, D), f32),
            pltpu.VMEM((_BPW,), jnp.int32),
            pltpu.VMEM((_BPW,), jnp.int32),
            pltpu.VMEM((_BPW,), jnp.int32),
            pltpu.SemaphoreType.DMA,
            pltpu.SemaphoreType.DMA,
            pltpu.SemaphoreType.DMA,
            pltpu.SemaphoreType.DMA,
        ],
    )
    return call(_sc_body)(
        user, pos, neg,
        user_table, item_table, user_bias_table, item_bias_table,
    )


def kernel(user, pos_item, neg_item, user_table, item_table,
           user_bias_table, item_bias_table):
    ue, pe, ne, ub, pb, nb = _bpr_lookup(
        user, pos_item, neg_item, user_table, item_table,
        user_bias_table, item_bias_table)
    return (ue, pe, ne,
            jnp.squeeze(ub, -1), jnp.squeeze(pb, -1), jnp.squeeze(nb, -1))


# speedup vs baseline: 1.6290x; 1.6290x over previous
import functools

import jax
import jax.numpy as jnp
from jax import lax
from jax.experimental import pallas as pl
from jax.experimental.pallas import tpu as pltpu
from jax.experimental.pallas import tpu_sc as plsc

B = 16384
D = 64

_info = plsc.get_sparse_core_info()
_NC, _NS, _L = _info.num_cores, _info.num_subcores, _info.num_lanes
_NW = _NC * _NS
_BPW = B // _NW
_CHUNK = 128
_NCHUNK = _BPW // _CHUNK


def _lane_take(x, idx):
    dnums = lax.GatherDimensionNumbers(
        offset_dims=(), collapsed_slice_dims=(0,), start_index_map=(0,))
    return lax.gather(x, idx[:, None], dnums, (1,),
                      mode=lax.GatherScatterMode.PROMISE_IN_BOUNDS)


def _normalize_rows(rows_ref, n_rows):
    lanes = lax.iota(jnp.int32, _L)
    perms = [lanes ^ sh for sh in (8, 4, 2, 1)]

    def body(r, carry):
        v0 = rows_ref[r, pl.ds(0, _L)]
        v1 = rows_ref[r, pl.ds(_L, _L)]
        v2 = rows_ref[r, pl.ds(2 * _L, _L)]
        v3 = rows_ref[r, pl.ds(3 * _L, _L)]
        ss = v0 * v0 + v1 * v1 + v2 * v2 + v3 * v3
        for p in perms:
            ss = ss + _lane_take(ss, p)
        s = ss
        i = lax.bitcast_convert_type(s, jnp.int32)
        y = lax.bitcast_convert_type(0x5F3759DF - (i >> 1), jnp.float32)
        nhalf = s * (-0.5)
        for _ in range(3):
            y = y * (1.5 + nhalf * y * y)
        rows_ref[r, pl.ds(0, _L)] = v0 * y
        rows_ref[r, pl.ds(_L, _L)] = v1 * y
        rows_ref[r, pl.ds(2 * _L, _L)] = v2 * y
        rows_ref[r, pl.ds(3 * _L, _L)] = v3 * y
        return carry

    lax.fori_loop(0, n_rows, body, 0)


def _one_list_body(idx_hbm, table, bias1d, out_emb, out_bias,
                   idx_v, rows_v, bias_v, sem_r, sem_b):
    wid = lax.axis_index("s") * _NC + lax.axis_index("c")
    base = wid * _BPW
    crow = wid * _NCHUNK

    pltpu.sync_copy(idx_hbm.at[pl.ds(crow, _NCHUNK)], idx_v)

    waits = []
    for j in range(_NCHUNK):
        dst = pl.ds(j * _CHUNK, _CHUNK)
        waits.append(pltpu.async_copy(
            table.at[idx_v.at[j]], rows_v.at[dst], sem_r))
        pltpu.async_copy(bias1d.at[idx_v.at[j]], bias_v.at[dst], sem_b)

    for w in waits:
        w.wait()
    _normalize_rows(rows_v, _BPW)
    pltpu.make_async_copy(bias1d.at[pl.ds(0, _BPW)], bias_v, sem_b).wait()

    obase = pl.ds(base, _BPW)
    pltpu.sync_copy(rows_v, out_emb.at[obase])
    pltpu.sync_copy(bias_v, out_bias.at[obase])


def _user_body(u_idx, user_table, ub1d, out_ue, out_ub,
               idx_v, rows_v, bias_v, sem_r, sem_b):
    _one_list_body(u_idx, user_table, ub1d, out_ue, out_ub,
                   idx_v, rows_v, bias_v, sem_r, sem_b)


def _item_body(p_idx, n_idx, item_table, ib1d,
               out_pe, out_ne, out_pb, out_nb,
               idx_v, rows_v, bias_v, sem_r, sem_b):
    _one_list_body(p_idx, item_table, ib1d, out_pe, out_pb,
                   idx_v, rows_v, bias_v, sem_r, sem_b)
    _one_list_body(n_idx, item_table, ib1d, out_ne, out_nb,
                   idx_v, rows_v, bias_v, sem_r, sem_b)


def _scratch(f32, i32):
    return [
        pltpu.VMEM((_NCHUNK, _CHUNK), i32),
        pltpu.VMEM((_BPW, D), f32),
        pltpu.VMEM((_BPW,), f32),
        pltpu.SemaphoreType.DMA,
        pltpu.SemaphoreType.DMA,
    ]


@jax.jit
def _bpr_lookup(user2d, pos2d, neg2d, user_table, item_table,
                user_bias_table, item_bias_table):
    mesh = plsc.VectorSubcoreMesh(core_axis_name="c", subcore_axis_name="s")
    f32 = jnp.float32
    i32 = jnp.int32
    params = pltpu.CompilerParams(use_tc_tiling_on_sc=False)

    ub1d = user_bias_table.reshape(-1)
    ib1d = item_bias_table.reshape(-1)

    user_call = functools.partial(
        pl.kernel, mesh=mesh, compiler_params=params,
        out_type=[
            jax.ShapeDtypeStruct((B, D), f32),
            jax.ShapeDtypeStruct((B,), f32),
        ],
        scratch_types=_scratch(f32, i32),
    )
    ue, ub = user_call(_user_body)(user2d, user_table, ub1d)

    item_call = functools.partial(
        pl.kernel, mesh=mesh, compiler_params=params,
        out_type=[
            jax.ShapeDtypeStruct((B, D), f32),
            jax.ShapeDtypeStruct((B, D), f32),
            jax.ShapeDtypeStruct((B,), f32),
            jax.ShapeDtypeStruct((B,), f32),
        ],
        scratch_types=_scratch(f32, i32),
    )
    pe, ne, pb, nb = item_call(_item_body)(pos2d, neg2d, item_table, ib1d)

    return ue, pe, ne, ub, pb, nb


def kernel(user, pos_item, neg_item, user_table, item_table,
           user_bias_table, item_bias_table):
    user2d = user.reshape(B // _CHUNK, _CHUNK)
    pos2d = pos_item.reshape(B // _CHUNK, _CHUNK)
    neg2d = neg_item.reshape(B // _CHUNK, _CHUNK)
    return _bpr_lookup(
        user2d, pos2d, neg2d, user_table, item_table,
        user_bias_table, item_bias_table)


# speedup vs baseline: 1.6501x; 1.0129x over previous
import functools

import jax
import jax.numpy as jnp
from jax import lax
from jax.experimental import pallas as pl
from jax.experimental.pallas import tpu as pltpu
from jax.experimental.pallas import tpu_sc as plsc

B = 16384
D = 64

_info = plsc.get_sparse_core_info()
_NC, _NS, _L = _info.num_cores, _info.num_subcores, _info.num_lanes
_NW = _NC * _NS
_BPW = B // _NW
_CHUNK = 128
_NCHUNK = _BPW // _CHUNK


def _lane_take(x, idx):
    dnums = lax.GatherDimensionNumbers(
        offset_dims=(), collapsed_slice_dims=(0,), start_index_map=(0,))
    return lax.gather(x, idx[:, None], dnums, (1,),
                      mode=lax.GatherScatterMode.PROMISE_IN_BOUNDS)


def _normalize_rows(rows_ref, n_rows):
    lanes = lax.iota(jnp.int32, _L)
    perms = [lanes ^ sh for sh in (8, 4, 2, 1)]

    def body(r, carry):
        v0 = rows_ref[r, pl.ds(0, _L)]
        v1 = rows_ref[r, pl.ds(_L, _L)]
        v2 = rows_ref[r, pl.ds(2 * _L, _L)]
        v3 = rows_ref[r, pl.ds(3 * _L, _L)]
        ss = v0 * v0 + v1 * v1 + v2 * v2 + v3 * v3
        for p in perms:
            ss = ss + _lane_take(ss, p)
        s = ss
        i = lax.bitcast_convert_type(s, jnp.int32)
        y = lax.bitcast_convert_type(0x5F3759DF - (i >> 1), jnp.float32)
        nhalf = s * (-0.5)
        for _ in range(3):
            y = y * (1.5 + nhalf * y * y)
        rows_ref[r, pl.ds(0, _L)] = v0 * y
        rows_ref[r, pl.ds(_L, _L)] = v1 * y
        rows_ref[r, pl.ds(2 * _L, _L)] = v2 * y
        rows_ref[r, pl.ds(3 * _L, _L)] = v3 * y
        return carry

    lax.fori_loop(0, n_rows, body, 0)


def _sc_body(
    u_idx_hbm, p_idx_hbm, n_idx_hbm,
    user_table, item_table, user_bias, item_bias,
    out_ue, out_pe, out_ne, out_ub, out_pb, out_nb,
    idx_u, idx_p, idx_n,
    rows_u, rows_p, rows_n,
    b_u, b_p, b_n,
    sem_u, sem_p, sem_n, sem_b,
):
    wid = lax.axis_index("s") * _NC + lax.axis_index("c")
    base = wid * _BPW
    crow = wid * _NCHUNK

    pltpu.sync_copy(u_idx_hbm.at[pl.ds(crow, _NCHUNK)], idx_u)
    pltpu.sync_copy(p_idx_hbm.at[pl.ds(crow, _NCHUNK)], idx_p)
    pltpu.sync_copy(n_idx_hbm.at[pl.ds(crow, _NCHUNK)], idx_n)

    waits_u, waits_p, waits_n, waits_b = [], [], [], []
    for j in range(_NCHUNK):
        dst = pl.ds(j * _CHUNK, _CHUNK)
        waits_u.append(pltpu.async_copy(
            user_table.at[idx_u.at[j]], rows_u.at[dst], sem_u))
        waits_p.append(pltpu.async_copy(
            item_table.at[idx_p.at[j]], rows_p.at[dst], sem_p))
        waits_n.append(pltpu.async_copy(
            item_table.at[idx_n.at[j]], rows_n.at[dst], sem_n))
        waits_b.append(pltpu.async_copy(
            user_bias.at[idx_u.at[j]], b_u.at[dst], sem_b))
        waits_b.append(pltpu.async_copy(
            item_bias.at[idx_p.at[j]], b_p.at[dst], sem_b))
        waits_b.append(pltpu.async_copy(
            item_bias.at[idx_n.at[j]], b_n.at[dst], sem_b))

    for w in waits_u:
        w.wait()
    _normalize_rows(rows_u, _BPW)
    for w in waits_p:
        w.wait()
    _normalize_rows(rows_p, _BPW)
    for w in waits_n:
        w.wait()
    _normalize_rows(rows_n, _BPW)
    for w in waits_b:
        w.wait()

    obase = pl.ds(base, _BPW)
    pltpu.sync_copy(rows_u, out_ue.at[obase])
    pltpu.sync_copy(rows_p, out_pe.at[obase])
    pltpu.sync_copy(rows_n, out_ne.at[obase])
    pltpu.sync_copy(b_u, out_ub.at[obase])
    pltpu.sync_copy(b_p, out_pb.at[obase])
    pltpu.sync_copy(b_n, out_nb.at[obase])


@jax.jit
def _bpr_lookup(user2d, pos2d, neg2d, user_table, item_table,
                user_bias_table, item_bias_table):
    mesh = plsc.VectorSubcoreMesh(core_axis_name="c", subcore_axis_name="s")
    f32 = jnp.float32
    call = functools.partial(
        pl.kernel,
        mesh=mesh,
        compiler_params=pltpu.CompilerParams(use_tc_tiling_on_sc=False),
        out_type=[
            jax.ShapeDtypeStruct((B, D), f32),
            jax.ShapeDtypeStruct((B, D), f32),
            jax.ShapeDtypeStruct((B, D), f32),
            jax.ShapeDtypeStruct((B,), f32),
            jax.ShapeDtypeStruct((B,), f32),
            jax.ShapeDtypeStruct((B,), f32),
        ],
        scratch_types=[
            pltpu.VMEM((_NCHUNK, _CHUNK), jnp.int32),
            pltpu.VMEM((_NCHUNK, _CHUNK), jnp.int32),
            pltpu.VMEM((_NCHUNK, _CHUNK), jnp.int32),
            pltpu.VMEM((_BPW, D), f32),
            pltpu.VMEM((_BPW, D), f32),
            pltpu.VMEM((_BPW, D), f32),
            pltpu.VMEM((_BPW,), f32),
            pltpu.VMEM((_BPW,), f32),
            pltpu.VMEM((_BPW,), f32),
            pltpu.SemaphoreType.DMA,
            pltpu.SemaphoreType.DMA,
            pltpu.SemaphoreType.DMA,
            pltpu.SemaphoreType.DMA,
        ],
    )
    return call(_sc_body)(
        user2d, pos2d, neg2d,
        user_table, item_table, user_bias_table, item_bias_table,
    )


def kernel(user, pos_item, neg_item, user_table, item_table,
           user_bias_table, item_bias_table):
    user2d = user.reshape(B // _CHUNK, _CHUNK)
    pos2d = pos_item.reshape(B // _CHUNK, _CHUNK)
    neg2d = neg_item.reshape(B // _CHUNK, _CHUNK)
    ue, pe, ne, ub, pb, nb = _bpr_lookup(
        user2d, pos2d, neg2d, user_table, item_table,
        user_bias_table.reshape(-1), item_bias_table.reshape(-1))
    return (ue, pe, ne, ub, pb, nb)
